# baseline (speedup 1.0000x reference)
import functools

import jax
import jax.numpy as jnp
from jax import lax
from jax.experimental import pallas as pl
from jax.experimental.pallas import tpu as pltpu
from jax.experimental.pallas import tpu_sc as plsc

_D = 64
_NC, _NS = 2, 16
_NW = _NC * _NS
_CHUNK = 128
_NCHUNK = 25
_W_ROWS = _CHUNK * _NCHUNK
_B_PAD = _NW * _W_ROWS


@functools.partial(
    pl.kernel,
    out_type=jax.ShapeDtypeStruct((_B_PAD, _D), jnp.float32),
    mesh=plsc.VectorSubcoreMesh(core_axis_name="c", subcore_axis_name="s"),
    scratch_types=[
        pltpu.VMEM((_NCHUNK, _CHUNK), jnp.int32),
        pltpu.VMEM((_CHUNK, _D), jnp.float32),
        pltpu.SemaphoreType.DMA,
    ],
    compiler_params=pltpu.CompilerParams(use_tc_tiling_on_sc=False),
)
def _emb_lookup(idx_hbm, table_hbm, out_hbm, idx_v, rows_v, gsem):
    wid = lax.axis_index("s") * _NC + lax.axis_index("c")
    base = wid * _W_ROWS
    pltpu.sync_copy(idx_hbm.at[wid], idx_v)

    def body(j, carry):
        pltpu.async_copy(table_hbm.at[idx_v.at[j]], rows_v, gsem).wait()
        pltpu.sync_copy(rows_v, out_hbm.at[pl.ds(base + j * _CHUNK, _CHUNK)])
        return carry

    lax.fori_loop(0, _NCHUNK, body, 0)


@jax.jit
def kernel(node_feature, weight):
    n = node_feature.shape[0]
    idx = node_feature[:, 0]
    idx_pad = jnp.pad(idx, (0, _B_PAD - n)).reshape(_NW, _NCHUNK, _CHUNK)
    out = _emb_lookup(idx_pad, weight)
    return out[:n]


# speedup vs baseline: 1.0585x; 1.0585x over previous
import functools

import jax
import jax.numpy as jnp
from jax import lax
from jax.experimental import pallas as pl
from jax.experimental.pallas import tpu as pltpu
from jax.experimental.pallas import tpu_sc as plsc

_D = 64
_NC, _NS = 2, 16
_NW = _NC * _NS
_CHUNK = 128
_NCHUNK = 25
_NBUF = 5
_NGROUP = _NCHUNK // _NBUF
_W_ROWS = _CHUNK * _NCHUNK
_B_PAD = _NW * _W_ROWS


@functools.partial(
    pl.kernel,
    out_type=jax.ShapeDtypeStruct((_B_PAD, _D), jnp.float32),
    mesh=plsc.VectorSubcoreMesh(core_axis_name="c", subcore_axis_name="s"),
    scratch_types=[
        pltpu.VMEM((_NCHUNK, _CHUNK), jnp.int32),
        pltpu.VMEM((_NBUF, _CHUNK, _D), jnp.float32),
        pltpu.SemaphoreType.DMA((_NBUF,)),
        pltpu.SemaphoreType.DMA((_NBUF,)),
    ],
    compiler_params=pltpu.CompilerParams(use_tc_tiling_on_sc=False),
)
def _emb_lookup(idx_hbm, table_hbm, out_hbm, idx_v, rows_v, gsem, ssem):
    wid = lax.axis_index("s") * _NC + lax.axis_index("c")
    base = wid * _W_ROWS
    pltpu.sync_copy(idx_hbm.at[wid], idx_v)

    def gather(j, b):
        pltpu.async_copy(table_hbm.at[idx_v.at[j]], rows_v.at[b], gsem.at[b])

    def gather_wait(b):
        pltpu.make_async_copy(
            table_hbm.at[idx_v.at[0]], rows_v.at[b], gsem.at[b]).wait()

    def store(j, b):
        pltpu.async_copy(
            rows_v.at[b], out_hbm.at[pl.ds(base + j * _CHUNK, _CHUNK)],
            ssem.at[b])

    def store_wait(b):
        pltpu.make_async_copy(
            out_hbm.at[pl.ds(base, _CHUNK)], rows_v.at[b], ssem.at[b]).wait()

    for b in range(_NBUF):
        gather(b, b)

    def group_body(g, carry):
        for b in range(_NBUF):
            gather_wait(b)
            store(g * _NBUF + b, b)

        @pl.when(g + 1 < _NGROUP)
        def _():
            for b in range(_NBUF):
                store_wait(b)
                gather((g + 1) * _NBUF + b, b)

        return carry

    lax.fori_loop(0, _NGROUP, group_body, 0)
    for b in range(_NBUF):
        store_wait(b)


@jax.jit
def kernel(node_feature, weight):
    n = node_feature.shape[0]
    idx = node_feature[:, 0]
    idx_pad = jnp.pad(idx, (0, _B_PAD - n)).reshape(_NW, _NCHUNK, _CHUNK)
    out = _emb_lookup(idx_pad, weight)
    return out[:n]


# speedup vs baseline: 1.6862x; 1.5930x over previous
import functools

import jax
import jax.numpy as jnp
from jax import lax
from jax.experimental import pallas as pl
from jax.experimental.pallas import tpu as pltpu
from jax.experimental.pallas import tpu_sc as plsc

_N = 100000
_D = 64
_NC, _NS = 2, 16
_NW = _NC * _NS
_CHUNK = 128
_NCHUNK = 25
_NBUF = 5
_NGROUP = _NCHUNK // _NBUF
_W_ROWS = _CHUNK * _NCHUNK


@functools.partial(
    pl.kernel,
    out_type=jax.ShapeDtypeStruct((_N, _D), jnp.float32),
    mesh=plsc.VectorSubcoreMesh(core_axis_name="c", subcore_axis_name="s"),
    scratch_types=[
        pltpu.VMEM((_W_ROWS,), jnp.int32),
        pltpu.VMEM((_NBUF, _CHUNK, _D), jnp.float32),
        pltpu.SemaphoreType.DMA((_NBUF,)),
        pltpu.SemaphoreType.DMA((_NBUF,)),
    ],
    compiler_params=pltpu.CompilerParams(use_tc_tiling_on_sc=False),
)
def _emb_lookup(idx_hbm, table_hbm, out_hbm, idx_v, rows_v, gsem, ssem):
    wid = lax.axis_index("s") * _NC + lax.axis_index("c")
    base = jnp.minimum(wid * _W_ROWS, _N - _W_ROWS)
    pltpu.sync_copy(idx_hbm.at[pl.ds(base, _W_ROWS)], idx_v)

    def gather(j, b):
        pltpu.async_copy(
            table_hbm.at[idx_v.at[pl.ds(j * _CHUNK, _CHUNK)]],
            rows_v.at[b], gsem.at[b])

    def gather_wait(b):
        pltpu.make_async_copy(
            table_hbm.at[idx_v.at[pl.ds(0, _CHUNK)]],
            rows_v.at[b], gsem.at[b]).wait()

    def store(j, b):
        pltpu.async_copy(
            rows_v.at[b], out_hbm.at[pl.ds(base + j * _CHUNK, _CHUNK)],
            ssem.at[b])

    def store_wait(b):
        pltpu.make_async_copy(
            out_hbm.at[pl.ds(base, _CHUNK)], rows_v.at[b], ssem.at[b]).wait()

    for b in range(_NBUF):
        gather(b, b)

    def group_body(g, carry):
        for b in range(_NBUF):
            gather_wait(b)
            store(g * _NBUF + b, b)

        @pl.when(g + 1 < _NGROUP)
        def _():
            for b in range(_NBUF):
                store_wait(b)
                gather((g + 1) * _NBUF + b, b)

        return carry

    lax.fori_loop(0, _NGROUP, group_body, 0)
    for b in range(_NBUF):
        store_wait(b)


@jax.jit
def kernel(node_feature, weight):
    return _emb_lookup(node_feature[:, 0], weight)


# speedup vs baseline: 1.6900x; 1.0023x over previous
import functools

import jax
import jax.numpy as jnp
from jax import lax
from jax.experimental import pallas as pl
from jax.experimental.pallas import tpu as pltpu
from jax.experimental.pallas import tpu_sc as plsc

_N = 100000
_D = 64
_NC, _NS = 2, 16
_NW = _NC * _NS
_CHUNK = 128
_NCHUNK = 25
_NBUF = 12
_LEAD = 8
_W_ROWS = _CHUNK * _NCHUNK


@functools.partial(
    pl.kernel,
    out_type=jax.ShapeDtypeStruct((_N, _D), jnp.float32),
    mesh=plsc.VectorSubcoreMesh(core_axis_name="c", subcore_axis_name="s"),
    scratch_types=[
        pltpu.VMEM((_W_ROWS,), jnp.int32),
        pltpu.VMEM((_NBUF, _CHUNK, _D), jnp.float32),
        pltpu.SemaphoreType.DMA((_NBUF,)),
        pltpu.SemaphoreType.DMA((_NBUF,)),
    ],
    compiler_params=pltpu.CompilerParams(use_tc_tiling_on_sc=False),
)
def _emb_lookup(idx_hbm, table_hbm, out_hbm, idx_v, rows_v, gsem, ssem):
    wid = lax.axis_index("s") * _NC + lax.axis_index("c")
    base = jnp.minimum(wid * _W_ROWS, _N - _W_ROWS)
    pltpu.sync_copy(idx_hbm.at[pl.ds(base, _W_ROWS)], idx_v)

    def gather(j, b):
        pltpu.async_copy(
            table_hbm.at[idx_v.at[pl.ds(j * _CHUNK, _CHUNK)]],
            rows_v.at[b], gsem.at[b])

    def gather_wait(b):
        pltpu.make_async_copy(
            table_hbm.at[idx_v.at[pl.ds(0, _CHUNK)]],
            rows_v.at[b], gsem.at[b]).wait()

    def store(j, b):
        pltpu.async_copy(
            rows_v.at[b], out_hbm.at[pl.ds(base + j * _CHUNK, _CHUNK)],
            ssem.at[b])

    def store_wait(b):
        pltpu.make_async_copy(
            out_hbm.at[pl.ds(base, _CHUNK)], rows_v.at[b], ssem.at[b]).wait()

    for q in range(_LEAD):
        gather(q, q % _NBUF)
    for m in range(_NCHUNK):
        q = m + _LEAD
        if q < _NCHUNK:
            bq = q % _NBUF
            if q >= _NBUF:
                store_wait(bq)
            gather(q, bq)
        b = m % _NBUF
        gather_wait(b)
        store(m, b)
    for m in range(max(0, _NCHUNK - _NBUF), _NCHUNK):
        store_wait(m % _NBUF)


@jax.jit
def kernel(node_feature, weight):
    return _emb_lookup(node_feature[:, 0], weight)
